# Initial kernel scaffold; baseline (speedup 1.0000x reference)
import jax
import jax.numpy as jnp
from jax.experimental import pallas as pl


def kernel(input_variable, embedding_weight):
    raise NotImplementedError("write your pallas kernel here")


# speedup vs baseline: 1.8594x; 1.8594x over previous
import functools

import jax
import jax.numpy as jnp
from jax import lax
from jax.experimental import pallas as pl
from jax.experimental.pallas import tpu as pltpu
from jax.experimental.pallas import tpu_sc as plsc

try:
    _info = plsc.get_sparse_core_info()
    NC, NS = _info.num_cores, _info.num_subcores
except Exception:
    NC, NS = 2, 16
NW = NC * NS

C = 128
NBUF = 8


@functools.cache
def _build(nchunk, D, interpret=False):
    mesh = plsc.VectorSubcoreMesh(
        core_axis_name="c", subcore_axis_name="s", num_cores=NC, num_subcores=NS
    )

    @functools.partial(
        pl.kernel,
        mesh=mesh,
        out_type=jax.ShapeDtypeStruct((NW, nchunk, C, D), jnp.float32),
        scratch_types=[
            pltpu.VMEM((nchunk, C), jnp.int32),
            *[pltpu.VMEM((C, D), jnp.float32) for _ in range(NBUF)],
            pltpu.SemaphoreType.DMA,
            pltpu.SemaphoreType.DMA,
        ],
        compiler_params=pltpu.CompilerParams(use_tc_tiling_on_sc=False),
        interpret=interpret,
    )
    def gather_kernel(idx_hbm, tab_hbm, out_hbm, idx_v, *rest):
        bufs = rest[:NBUF]
        sem_g, sem_s = rest[NBUF], rest[NBUF + 1]
        wid = lax.axis_index("s") * NC + lax.axis_index("c")
        pltpu.sync_copy(idx_hbm.at[wid], idx_v)

        def group(g, carry):
            gathers = []
            for b in range(NBUF):
                j = g * NBUF + b
                gathers.append(
                    pltpu.async_copy(tab_hbm.at[idx_v.at[j]], bufs[b], sem_g)
                )
            for h in gathers:
                h.wait()
            stores = []
            for b in range(NBUF):
                j = g * NBUF + b
                stores.append(
                    pltpu.async_copy(bufs[b], out_hbm.at[wid, j], sem_s)
                )
            for h in stores:
                h.wait()
            return carry

        lax.fori_loop(0, nchunk // NBUF, group, 0, unroll=False)

    return gather_kernel


def kernel(input_variable, embedding_weight):
    B, H = input_variable.shape
    V, D = embedding_weight.shape
    total = B * H
    assert total % (NW * C) == 0
    nchunk = total // (NW * C)
    assert nchunk % NBUF == 0
    idx = input_variable.reshape(NW, nchunk, C).astype(jnp.int32)
    out = _build(nchunk, D)(idx, embedding_weight)
    return out.reshape(B, H, D)


# speedup vs baseline: 1.8736x; 1.0076x over previous
import functools

import jax
import jax.numpy as jnp
from jax import lax
from jax.experimental import pallas as pl
from jax.experimental.pallas import tpu as pltpu
from jax.experimental.pallas import tpu_sc as plsc

try:
    _info = plsc.get_sparse_core_info()
    NC, NS = _info.num_cores, _info.num_subcores
except Exception:
    NC, NS = 2, 16
NW = NC * NS

C = 128
S = 10
L = 5


@functools.cache
def _build(nchunk, D):
    mesh = plsc.VectorSubcoreMesh(
        core_axis_name="c", subcore_axis_name="s", num_cores=NC, num_subcores=NS
    )
    ngroups = nchunk // S

    @functools.partial(
        pl.kernel,
        mesh=mesh,
        out_type=jax.ShapeDtypeStruct((NW, nchunk, C, D), jnp.float32),
        scratch_types=[
            pltpu.VMEM((nchunk, C), jnp.int32),
            *[pltpu.VMEM((C, D), jnp.float32) for _ in range(S)],
            *[pltpu.SemaphoreType.DMA for _ in range(2 * S)],
        ],
        compiler_params=pltpu.CompilerParams(use_tc_tiling_on_sc=False),
    )
    def gather_kernel(idx_hbm, tab_hbm, out_hbm, idx_v, *rest):
        bufs = rest[:S]
        sem_g = rest[S : 2 * S]
        sem_s = rest[2 * S : 3 * S]
        wid = lax.axis_index("s") * NC + lax.axis_index("c")
        pltpu.sync_copy(idx_hbm.at[wid], idx_v)

        def fire_gather(j, b):
            pltpu.async_copy(tab_hbm.at[idx_v.at[j]], bufs[b], sem_g[b])

        def wait_gather(j, b):
            pltpu.make_async_copy(tab_hbm.at[idx_v.at[j]], bufs[b], sem_g[b]).wait()

        def fire_store(j, b):
            pltpu.async_copy(bufs[b], out_hbm.at[wid, j], sem_s[b])

        def wait_store(j, b):
            pltpu.make_async_copy(bufs[b], out_hbm.at[wid, j], sem_s[b]).wait()

        for b in range(L):
            fire_gather(b, b)

        for b in range(S):
            wait_gather(b, b)
            fire_store(b, b)
            jn, bn = b + L, (b + L) % S
            if jn >= S:
                wait_store(jn - S, bn)
            fire_gather(jn, bn)

        def group(g, carry):
            base = g * S
            for b in range(S):
                j = base + b
                wait_gather(j, b)
                fire_store(j, b)
                jn, bn = j + L, (b + L) % S
                wait_store(jn - S, bn)
                fire_gather(jn, bn)
            return carry

        lax.fori_loop(1, ngroups - 1, group, 0, unroll=False)

        base = (ngroups - 1) * S
        for b in range(S):
            j = base + b
            wait_gather(j, b)
            fire_store(j, b)
            jn, bn = j + L, (b + L) % S
            if jn < nchunk:
                wait_store(jn - S, bn)
                fire_gather(jn, bn)

        for b in range(S):
            wait_store(base + b, b)

    return gather_kernel


def kernel(input_variable, embedding_weight):
    B, H = input_variable.shape
    V, D = embedding_weight.shape
    total = B * H
    assert total % (NW * C) == 0
    nchunk = total // (NW * C)
    assert nchunk % S == 0 and nchunk // S >= 2
    idx = input_variable.reshape(NW, nchunk, C).astype(jnp.int32)
    out = _build(nchunk, D)(idx, embedding_weight)
    return out.reshape(B, H, D)
